# baseline (speedup 1.0000x reference)
import functools

import jax
import jax.numpy as jnp
from jax import lax
from jax.experimental import pallas as pl
from jax.experimental.pallas import tpu as pltpu
from jax.experimental.pallas import tpu_sc as plsc

_NC = 2
_NS = 16


@functools.lru_cache(maxsize=None)
def _build(seq: int, nb: int, dim: int, vocab: int):
    half = nb // _NC
    n_full = seq // _NS
    rem = seq - n_full * _NS

    def body(xt_hbm, et_hbm, out_hbm, idx0, idx1, g0, g1, spm,
             sem_i0, sem_i1, sem_g0, sem_g1, sem_w0, sem_w1):
        ci = lax.axis_index("c")
        si = lax.axis_index("s")
        b0 = ci * half
        idxb = (idx0, idx1)
        gb = (g0, g1)
        sem_i = (sem_i0, sem_i1)
        sem_g = (sem_g0, sem_g1)
        sem_w = (sem_w0, sem_w1)

        nr = n_full + jnp.where(si < rem, 1, 0)

        def row_of(k):
            return si + k * _NS

        def idx_start(k, p):
            pltpu.async_copy(xt_hbm.at[row_of(k), pl.ds(b0, half)],
                             idxb[p], sem_i[p])

        def idx_wait(k, p):
            pltpu.make_async_copy(xt_hbm.at[row_of(k), pl.ds(b0, half)],
                                  idxb[p], sem_i[p]).wait()

        def w_start(c, k, p):
            pltpu.async_copy(gb[p], out_hbm.at[row_of(k), c, pl.ds(b0, half)],
                             sem_w[p])

        def w_wait(c, k, p):
            pltpu.make_async_copy(gb[p],
                                  out_hbm.at[row_of(k), c, pl.ds(b0, half)],
                                  sem_w[p]).wait()

        @pl.loop(0, dim)
        def _(c):
            @pl.when(si == 0)
            def _():
                pltpu.sync_copy(et_hbm.at[c], spm)

            plsc.subcore_barrier()

            idx_start(0, 0)
            for k in range(n_full + 1):
                p = k % 2

                @pl.when(k < nr)
                def _():
                    idx_wait(k, p)

                    @pl.when(k + 1 < nr)
                    def _():
                        idx_start(k + 1, 1 - p)

                    ng = half // 16

                    @pl.loop(0, ng)
                    def _(g):
                        iv = idxb[p][pl.ds(g * 16, 16)]
                        pltpu.async_copy(spm.at[iv],
                                         gb[p].at[pl.ds(g * 16, 16)],
                                         sem_g[p])

                        @pl.when(g >= 16)
                        def _():
                            iw = idxb[p][pl.ds((g - 16) * 16, 16)]
                            pltpu.make_async_copy(
                                spm.at[iw],
                                gb[p].at[pl.ds((g - 16) * 16, 16)],
                                sem_g[p]).wait()

                    @pl.loop(ng - 16, ng)
                    def _(g):
                        iw = idxb[p][pl.ds(g * 16, 16)]
                        pltpu.make_async_copy(
                            spm.at[iw],
                            gb[p].at[pl.ds(g * 16, 16)], sem_g[p]).wait()

                    @pl.when(k >= 2)
                    def _():
                        w_wait(c, k - 2, p)

                    w_start(c, k, p)

            for k in (n_full - 1, n_full):
                @pl.when(k == nr - 1)
                def _():
                    w_wait(c, k - 1, (k - 1) % 2)
                    w_wait(c, k, k % 2)

            plsc.subcore_barrier()

    return pl.kernel(
        body,
        out_type=jax.ShapeDtypeStruct((seq, dim, nb), jnp.float32),
        compiler_params=pltpu.CompilerParams(use_tc_tiling_on_sc=True),
        mesh=plsc.VectorSubcoreMesh(core_axis_name="c", subcore_axis_name="s"),
        scratch_types=[
            pltpu.VMEM((half,), jnp.int32),
            pltpu.VMEM((half,), jnp.int32),
            pltpu.VMEM((half,), jnp.float32),
            pltpu.VMEM((half,), jnp.float32),
            pltpu.VMEM_SHARED((vocab,), jnp.float32),
            pltpu.SemaphoreType.DMA,
            pltpu.SemaphoreType.DMA,
            pltpu.SemaphoreType.DMA,
            pltpu.SemaphoreType.DMA,
            pltpu.SemaphoreType.DMA,
            pltpu.SemaphoreType.DMA,
        ],
    )


def kernel(x, emb_weight):
    vocab, dim = emb_weight.shape
    nb, seq = x.shape
    xt = x.T
    et = emb_weight.T
    out_t = _build(seq, nb, dim, vocab)(xt, et)
    return (jnp.transpose(out_t, (2, 0, 1)), 0.0)


# speedup vs baseline: 2.3245x; 2.3245x over previous
import functools

import jax
import jax.numpy as jnp
from jax import lax
from jax.experimental import pallas as pl
from jax.experimental.pallas import tpu as pltpu
from jax.experimental.pallas import tpu_sc as plsc

_NC = 2
_NS = 16


@functools.lru_cache(maxsize=None)
def _build(seq: int, nb: int, dim: int, vocab: int):
    half = nb // _NC
    n_full = seq // _NS
    rem = seq - n_full * _NS

    def body(xt_hbm, et_hbm, out_hbm, idx0, idx1, g0, g1, spm,
             sem_i0, sem_i1, sem_g0, sem_g1, sem_w0, sem_w1):
        ci = lax.axis_index("c")
        si = lax.axis_index("s")
        b0 = ci * half
        idxb = (idx0, idx1)
        gb = (g0, g1)
        sem_i = (sem_i0, sem_i1)
        sem_g = (sem_g0, sem_g1)
        sem_w = (sem_w0, sem_w1)

        nr = n_full + jnp.where(si < rem, 1, 0)

        def row_of(k):
            return si + k * _NS

        def idx_start(k, p):
            pltpu.async_copy(xt_hbm.at[row_of(k), pl.ds(b0, half)],
                             idxb[p], sem_i[p])

        def idx_wait(k, p):
            pltpu.make_async_copy(xt_hbm.at[row_of(k), pl.ds(b0, half)],
                                  idxb[p], sem_i[p]).wait()

        def w_start(c, k, p):
            pltpu.async_copy(gb[p], out_hbm.at[row_of(k), c, pl.ds(b0, half)],
                             sem_w[p])

        def w_wait(c, k, p):
            pltpu.make_async_copy(gb[p],
                                  out_hbm.at[row_of(k), c, pl.ds(b0, half)],
                                  sem_w[p]).wait()

        @pl.loop(0, dim)
        def _(c):
            @pl.when(si == 0)
            def _():
                pltpu.sync_copy(et_hbm.at[c], spm)

            plsc.subcore_barrier()

            idx_start(0, 0)
            for k in range(n_full + 1):
                p = k % 2

                @pl.when(k < nr)
                def _():
                    idx_wait(k, p)

                    @pl.when(k + 1 < nr)
                    def _():
                        idx_start(k + 1, 1 - p)

                    pltpu.async_copy(spm.at[idxb[p]], gb[p], sem_g[p])
                    pltpu.make_async_copy(spm.at[idxb[p]], gb[p],
                                          sem_g[p]).wait()

                    @pl.when(k >= 2)
                    def _():
                        w_wait(c, k - 2, p)

                    w_start(c, k, p)

            for k in (n_full - 1, n_full):
                @pl.when(k == nr - 1)
                def _():
                    w_wait(c, k - 1, (k - 1) % 2)
                    w_wait(c, k, k % 2)

            plsc.subcore_barrier()

    return pl.kernel(
        body,
        out_type=jax.ShapeDtypeStruct((seq, dim, nb), jnp.float32),
        compiler_params=pltpu.CompilerParams(use_tc_tiling_on_sc=True),
        mesh=plsc.VectorSubcoreMesh(core_axis_name="c", subcore_axis_name="s"),
        scratch_types=[
            pltpu.VMEM((half,), jnp.int32),
            pltpu.VMEM((half,), jnp.int32),
            pltpu.VMEM((half,), jnp.float32),
            pltpu.VMEM((half,), jnp.float32),
            pltpu.VMEM_SHARED((vocab,), jnp.float32),
            pltpu.SemaphoreType.DMA,
            pltpu.SemaphoreType.DMA,
            pltpu.SemaphoreType.DMA,
            pltpu.SemaphoreType.DMA,
            pltpu.SemaphoreType.DMA,
            pltpu.SemaphoreType.DMA,
        ],
    )


def kernel(x, emb_weight):
    vocab, dim = emb_weight.shape
    nb, seq = x.shape
    xt = x.T
    et = emb_weight.T
    out_t = _build(seq, nb, dim, vocab)(xt, et)
    return (jnp.transpose(out_t, (2, 0, 1)), 0.0)
